# baseline (speedup 1.0000x reference)
import functools

import jax
import jax.numpy as jnp
from jax import lax
from jax.experimental import pallas as pl
from jax.experimental.pallas import tpu as pltpu
from jax.experimental.pallas import tpu_sc as plsc

N = 10000
E = 320000
D = 128
G = 128

NC = 2
NS = 16
NW = NC * NS
CH = 64
NCH = 162
EPT = NCH * CH
EPAD = NW * EPT
NR = 3
NI = 6
NGRP = NCH // NI
NPAD = 10240
ZPT = NPAD // NS
ZCH = ZPT // CH

_sc_mesh = plsc.VectorSubcoreMesh(
    core_axis_name="c", subcore_axis_name="s", num_cores=NC, num_subcores=NS)


@functools.partial(
    pl.kernel,
    out_type=jax.ShapeDtypeStruct((NC, NPAD, D), jnp.float32),
    mesh=_sc_mesh,
    scratch_types=[
        pltpu.VMEM((NI, CH), jnp.int32),
        pltpu.VMEM((NR, CH), jnp.int32),
        pltpu.VMEM((NR, CH, D), jnp.float32),
        pltpu.VMEM_SHARED((NPAD, D), jnp.float32),
    ] + [pltpu.SemaphoreType.DMA] * (NI + 3 * NR),
)
def _edge_agg(h_hbm, src_hbm, dst_hbm, out_hbm, sring, dring, rows_v,
              acc_sh, *sems):
    is_sem = sems[:NI]
    id_sem = sems[NI:NI + NR]
    gsem = sems[NI + NR:NI + 2 * NR]
    ssem = sems[NI + 2 * NR:]
    cid = lax.axis_index("c")
    sid = lax.axis_index("s")
    wid = sid * NC + cid
    base = wid * EPT

    def issue_src(c, a):
        pltpu.async_copy(src_hbm.at[pl.ds(base + c * CH, CH)], sring.at[a],
                         is_sem[a])

    def issue_dst(c, r):
        pltpu.async_copy(dst_hbm.at[pl.ds(base + c * CH, CH)], dring.at[r],
                         id_sem[r])

    def wait_src(a):
        pltpu.make_async_copy(src_hbm.at[pl.ds(0, CH)], sring.at[a],
                              is_sem[a]).wait()

    def wait_dst(r):
        pltpu.make_async_copy(dst_hbm.at[pl.ds(0, CH)], dring.at[r],
                              id_sem[r]).wait()

    def issue_gather(a, r):
        pltpu.async_copy(h_hbm.at[sring.at[a]], rows_v.at[r], gsem[r])

    def wait_gather(r):
        pltpu.make_async_copy(h_hbm.at[sring.at[0]], rows_v.at[r],
                              gsem[r]).wait()

    def issue_scatter(r):
        pltpu.async_copy(rows_v.at[r], acc_sh.at[dring.at[r]], ssem[r],
                         add=True)

    def wait_scatter(r):
        pltpu.make_async_copy(rows_v.at[r], acc_sh.at[dring.at[r]],
                              ssem[r]).wait()

    for c in range(NI):
        issue_src(c, c % NI)
    for c in range(NR):
        issue_dst(c, c % NR)

    zeros16 = jnp.zeros((16,), jnp.float32)

    @pl.loop(0, CH)
    def _zero_rows(rr):
        @pl.loop(0, D // 16)
        def _zero_cols(cc):
            rows_v[0, rr, pl.ds(cc * 16, 16)] = zeros16

    @pl.loop(0, ZCH)
    def _zero_acc(z):
        pltpu.sync_copy(rows_v.at[0], acc_sh.at[pl.ds(sid * ZPT + z * CH, CH)])

    plsc.subcore_barrier()

    for c in range(NI):
        r, a = c % NR, c % NI
        r2 = (c - 2) % NR
        if c >= NR:
            wait_scatter(r)
            issue_dst(c, r)
        wait_src(a)
        issue_gather(a, r)
        if c >= 2:
            wait_gather(r2)
            issue_src(c + 4, (c + 4) % NI)
            wait_dst(r2)
            issue_scatter(r2)

    @pl.loop(1, NGRP - 1)
    def _groups(g):
        c0 = g * NI
        for u in range(NI):
            c = c0 + u
            r, a = u % NR, u % NI
            r2, a4 = (u - 2) % NR, (u + 4) % NI
            wait_scatter(r)
            issue_dst(c, r)
            wait_src(a)
            issue_gather(a, r)
            wait_gather(r2)
            issue_src(c + 4, a4)
            wait_dst(r2)
            issue_scatter(r2)

    for c in range(NCH - NI, NCH):
        u = c % NI
        r, a = u % NR, u % NI
        r2 = (u - 2) % NR
        wait_scatter(r)
        issue_dst(c, r)
        wait_src(a)
        issue_gather(a, r)
        wait_gather(r2)
        if c + 4 < NCH:
            issue_src(c + 4, (c + 4) % NI)
        wait_dst(r2)
        issue_scatter(r2)

    for c in range(NCH, NCH + 2):
        r2 = (c - 2) % NR
        wait_gather(r2)
        wait_dst(r2)
        issue_scatter(r2)
    for r in range(NR):
        wait_scatter(r)

    plsc.subcore_barrier()

    pltpu.sync_copy(acc_sh.at[pl.ds(sid * ZPT, ZPT)],
                    out_hbm.at[cid, pl.ds(sid * ZPT, ZPT)])


BN = 2000
NB = N // BN


def _layer_body(h_ref, p0_ref, p1_ref, w_ref, b_ref, o_ref):
    s = h_ref[...] + p0_ref[...] + p1_ref[...]
    y = jnp.dot(s, w_ref[...], preferred_element_type=jnp.float32) + b_ref[...]
    o_ref[...] = jnp.maximum(y, 0.0)


def _layer_tc(h, p0, p1, W, b2d):
    return pl.pallas_call(
        _layer_body,
        grid=(NB,),
        in_specs=[
            pl.BlockSpec((BN, D), lambda i: (i, 0)),
            pl.BlockSpec((BN, D), lambda i: (i, 0)),
            pl.BlockSpec((BN, D), lambda i: (i, 0)),
            pl.BlockSpec((D, D), lambda i: (0, 0)),
            pl.BlockSpec((1, D), lambda i: (0, 0)),
        ],
        out_specs=pl.BlockSpec((BN, D), lambda i: (i, 0)),
        out_shape=jax.ShapeDtypeStruct((N, D), jnp.float32),
    )(h, p0, p1, W, b2d)


def _final_body(h_ref, p0_ref, p1_ref, w2_ref, b2_ref, batch_ref, wg_ref,
                bg_ref, o_ref, sums, counts):
    i = pl.program_id(0)

    @pl.when(i == 0)
    def _():
        sums[...] = jnp.zeros_like(sums)
        counts[...] = jnp.zeros_like(counts)

    s = h_ref[...] + p0_ref[...] + p1_ref[...]
    h2 = jnp.maximum(
        jnp.dot(s, w2_ref[...], preferred_element_type=jnp.float32)
        + b2_ref[...], 0.0)

    bt = batch_ref[...].reshape(1, BN)
    gidx = lax.broadcasted_iota(jnp.int32, (G, BN), 0)
    P = (bt == gidx).astype(jnp.float32)
    sums[...] += jnp.dot(P, h2, preferred_element_type=jnp.float32)
    counts[...] += jnp.broadcast_to(jnp.sum(P, axis=1, keepdims=True), (G, D))

    @pl.when(i == NB - 1)
    def _():
        hg = sums[...] / jnp.maximum(counts[...], 1.0)
        o_ref[...] = (jnp.dot(hg, wg_ref[...], preferred_element_type=jnp.float32)
                      + bg_ref[...])


def _final_tc(h1, p0, p1, W2, b2d, batch3d, Wg, bg2d):
    return pl.pallas_call(
        _final_body,
        grid=(NB,),
        in_specs=[
            pl.BlockSpec((BN, D), lambda i: (i, 0)),
            pl.BlockSpec((BN, D), lambda i: (i, 0)),
            pl.BlockSpec((BN, D), lambda i: (i, 0)),
            pl.BlockSpec((D, D), lambda i: (0, 0)),
            pl.BlockSpec((1, D), lambda i: (0, 0)),
            pl.BlockSpec((1, 1, BN), lambda i: (i, 0, 0)),
            pl.BlockSpec((D, D), lambda i: (0, 0)),
            pl.BlockSpec((1, D), lambda i: (0, 0)),
        ],
        out_specs=pl.BlockSpec((G, D), lambda i: (0, 0)),
        out_shape=jax.ShapeDtypeStruct((G, D), jnp.float32),
        scratch_shapes=[
            pltpu.VMEM((G, D), jnp.float32),
            pltpu.VMEM((G, D), jnp.float32),
        ],
    )(h1, p0, p1, W2, b2d, batch3d, Wg, bg2d)


def kernel(x, edge_index, batch, W1, b1, W2, b2, Wg, bg):
    pad = EPAD - E
    src = jnp.concatenate(
        [edge_index[0].astype(jnp.int32), jnp.zeros((pad,), jnp.int32)])
    dst = jnp.concatenate(
        [edge_index[1].astype(jnp.int32),
         jnp.full((pad,), NPAD - 1, jnp.int32)])
    batch3d = batch.astype(jnp.int32).reshape(NB, 1, BN)

    p = _edge_agg(x, src, dst)
    h1 = _layer_tc(x, p[0, :N], p[1, :N], W1, b1.reshape(1, D))
    q = _edge_agg(h1, src, dst)
    return _final_tc(h1, q[0, :N], q[1, :N], W2, b2.reshape(1, D),
                     batch3d, Wg, bg.reshape(1, D))
